# baseline (speedup 1.0000x reference)
import functools

import jax
import jax.numpy as jnp
from jax import lax
from jax.experimental import pallas as pl
from jax.experimental.pallas import tpu as pltpu
from jax.experimental.pallas import tpu_sc as plsc

BATCH = 16384
DIM = 16
NC, NS = 2, 16
NW = NC * NS
L = 16
NROWS = 1000000
MAIN_HI = 999936
ROWS_PER_W = 31360
CW = 1152
TAILW = 64
OUT_ROWS = BATCH + 128
VCAP = 288
SEGCAP = 160
NGRP = BATCH // L


def _sc_body(ta_u, tb_u, ta_i, tb_i, user_h, item_h, gu_out, gi_out,
             idx_v, wr, wp, cha, chb, tha, thb, segc, segp,
             valbuf, posb, flidx, sem):
    wid = lax.axis_index("s") * NC + lax.axis_index("c")
    lane = lax.iota(jnp.int32, L)
    lo_w = wid * ROWS_PER_W
    hi_w = jnp.minimum(lo_w + ROWS_PER_W, MAIN_HI)
    is_last = wid == NW - 1
    hi_ext = jnp.where(is_last, NROWS, hi_w)
    nch = (hi_w - lo_w + CW - 1) // CW

    def zero_body(r, _):
        for j in range(8):
            valbuf[r, pl.ds(16 * j, L)] = jnp.zeros((L,), jnp.float32)
        return 0
    lax.fori_loop(0, VCAP, zero_body, 0)

    def extract_group(refa, refb, colv, posv, rowv, m):
        plsc.store_scatter(posb, [rowv], posv, mask=m)
        for f in range(DIM):
            fv = jnp.full((L,), f, jnp.int32)
            va = plsc.load_gather(refa, [fv, colv], mask=m)
            plsc.store_scatter(valbuf, [rowv, fv], va, mask=m)
            vb = plsc.load_gather(refb, [fv, colv], mask=m)
            plsc.store_scatter(valbuf, [rowv, fv + DIM], vb, mask=m)

    def flush(nb, out):
        for j in range(8):
            pr = posb[pl.ds(16 * j, L)]
            lp = 16 * j + lane
            pr = jnp.where(lp < nb, pr, BATCH + lp)
            flidx[0, pl.ds(16 * j, L)] = pr
        pltpu.async_copy(valbuf.at[pl.ds(0, 128)], out.at[flidx.at[0]],
                         sem).wait()
        mv = jnp.maximum(nb - 128, 0)

        def mv_pos(j, _):
            posb[pl.ds(16 * j, L)] = posb[pl.ds(128 + 16 * j, L)]
            return 0
        lax.fori_loop(0, (mv + 15) // 16, mv_pos, 0)

        def mv_val(r, _):
            for j in range(8):
                valbuf[r, pl.ds(16 * j, L)] = valbuf[128 + r, pl.ds(16 * j, L)]
            return 0
        lax.fori_loop(0, mv, mv_val, 0)
        return mv

    def process_range(refa, refb, c0, rlo, rhi, nb, n_w, out):
        def drain(sf, nb):
            for j in range(8):
                colv = segc[pl.ds(16 * j, L)]
                posv = segp[pl.ds(16 * j, L)]
                rowv = nb + 16 * j + lane
                extract_group(refa, refb, colv, posv, rowv,
                              jnp.ones((L,), jnp.bool_))
            segc[pl.ds(0, L)] = segc[pl.ds(128, L)]
            segp[pl.ds(0, L)] = segp[pl.ds(128, L)]
            nb = flush(nb + 128, out)
            return sf - 128, nb

        def g_body(g, st):
            sf, nb = st
            rv = wr[pl.ds(16 * g, L)]
            pv = wp[pl.ds(16 * g, L)]
            m = (lane < (n_w - 16 * g)) & (rv >= rlo) & (rv < rhi)
            mi = jnp.where(m, 1, 0)
            tgt = sf + plsc.cumsum(mi) - mi
            plsc.store_scatter(segc, [tgt], jnp.where(m, rv - c0, 0), mask=m)
            plsc.store_scatter(segp, [tgt], pv, mask=m)
            sf = sf + jnp.sum(mi)
            return lax.cond(sf >= 128, drain, lambda a, b: (a, b), sf, nb)

        ngw = (n_w + L - 1) // L
        sf, nb = lax.fori_loop(0, ngw, g_body, (0, nb))

        def tail_grp(j, nb2):
            colv = segc[pl.ds(16 * j, L)]
            posv = segp[pl.ds(16 * j, L)]
            m = lane < (sf - 16 * j)
            extract_group(refa, refb, jnp.where(m, colv, 0), posv,
                          nb2 + 16 * j + lane, m)
            return nb2
        lax.fori_loop(0, (sf + L - 1) // L, tail_grp, nb)
        return nb + sf

    for side in range(2):
        refa, refb = ((ta_u, tb_u), (ta_i, tb_i))[side]
        idx_h = (user_h, item_h)[side]
        out = (gu_out, gi_out)[side]
        pltpu.sync_copy(idx_h, idx_v)

        def build(g, nw):
            iv = idx_v[pl.ds(16 * g, L)]
            m = (iv >= lo_w) & (iv < hi_ext)
            mi = jnp.where(m, 1, 0)
            tgt = nw + plsc.cumsum(mi) - mi
            plsc.store_scatter(wr, [tgt], iv, mask=m)
            plsc.store_scatter(wp, [tgt], 16 * g + lane, mask=m)
            return nw + jnp.sum(mi)
        n_w = lax.fori_loop(0, NGRP, build, 0)

        def chunk_body(k, nb):
            c0 = jnp.minimum(lo_w + k * CW, hi_w - CW)
            rlo = lo_w + k * CW
            rhi = jnp.minimum(rlo + CW, hi_w)
            pltpu.sync_copy(refa.at[:, pl.ds(c0, CW)], cha)
            pltpu.sync_copy(refb.at[:, pl.ds(c0, CW)], chb)
            nb = process_range(cha, chb, c0, rlo, rhi, nb, n_w, out)
            return lax.while_loop(lambda n: n >= 128,
                                  lambda n: flush(n, out), nb)
        nb = lax.fori_loop(0, nch, chunk_body, 0)

        def tail_fn(nb):
            pltpu.sync_copy(refa.at[:, pl.ds(MAIN_HI, TAILW)], tha)
            pltpu.sync_copy(refb.at[:, pl.ds(MAIN_HI, TAILW)], thb)
            return process_range(tha, thb, MAIN_HI, MAIN_HI, NROWS,
                                 nb, n_w, out)
        nb = lax.cond(is_last, tail_fn, lambda n: n, nb)

        lax.while_loop(lambda n: n > 0, lambda n: flush(n, out), nb)


@functools.cache
def _sc_gather():
    return pl.kernel(
        _sc_body,
        out_type=[jax.ShapeDtypeStruct((OUT_ROWS, 128), jnp.float32)] * 2,
        mesh=plsc.VectorSubcoreMesh(core_axis_name="c", subcore_axis_name="s",
                                    num_cores=NC, num_subcores=NS),
        scratch_types=[
            pltpu.VMEM((BATCH,), jnp.int32),
            pltpu.VMEM((BATCH,), jnp.int32),
            pltpu.VMEM((BATCH,), jnp.int32),
            pltpu.VMEM((16, CW), jnp.float32),
            pltpu.VMEM((16, CW), jnp.float32),
            pltpu.VMEM((16, TAILW), jnp.float32),
            pltpu.VMEM((16, TAILW), jnp.float32),
            pltpu.VMEM((SEGCAP,), jnp.int32),
            pltpu.VMEM((SEGCAP,), jnp.int32),
            pltpu.VMEM((VCAP, 128), jnp.float32),
            pltpu.VMEM((VCAP,), jnp.int32),
            pltpu.VMEM((1, 128), jnp.int32),
            pltpu.SemaphoreType.DMA,
        ],
        compiler_params=pltpu.CompilerParams(needs_layout_passes=False),
    )


BLK = 2048


def _tc_mlp_body(gu, gi, a, b, b1, w2, b2, wpm, wph, bp, out):
    u = gu[...]
    v = gi[...]
    lmf = jnp.sum(u * v * wpm[...], axis=1)
    h = jnp.dot(u, a[...], preferred_element_type=jnp.float32)
    h += jnp.dot(v, b[...], preferred_element_type=jnp.float32)
    h = jnp.maximum(h + b1[...], 0.0)
    h = jnp.dot(h, w2[...], preferred_element_type=jnp.float32)
    h = jnp.maximum(h + b2[...], 0.0)
    logit = lmf + jnp.sum(h * wph[...], axis=1) + bp[0, 0]
    out[...] = 1.0 / (1.0 + jnp.exp(-logit))


def _tc_mlp(gu, gi, a, b, b1, w2, b2, wpm, wph, bp):
    nblk = BATCH // BLK
    row_blk = pl.BlockSpec((BLK, 128), lambda i: (i, 0))
    full = lambda x: pl.BlockSpec(x.shape, lambda i: (0,) * x.ndim)
    return pl.pallas_call(
        _tc_mlp_body,
        grid=(nblk,),
        in_specs=[row_blk, row_blk, full(a), full(b), full(b1), full(w2),
                  full(b2), full(wpm), full(wph), full(bp)],
        out_specs=pl.BlockSpec((BLK,), lambda i: (i,)),
        out_shape=jax.ShapeDtypeStruct((BATCH,), jnp.float32),
        compiler_params=pltpu.CompilerParams(
            dimension_semantics=("arbitrary",)),
    )(gu, gi, a, b, b1, w2, b2, wpm, wph, bp)


def kernel(user, item, U_mf, I_mf, U_mlp, I_mlp, W1, b1, W2, b2, Wp, bp):
    gu, gi = _sc_gather()(U_mf.T, U_mlp.T, I_mf.T, I_mlp.T, user, item)
    a = jnp.zeros((128, DIM), jnp.float32).at[DIM:2 * DIM, :].set(
        W1[:, :DIM].T)
    b = jnp.zeros((128, DIM), jnp.float32).at[DIM:2 * DIM, :].set(
        W1[:, DIM:].T)
    wpm = jnp.zeros((1, 128), jnp.float32).at[0, :DIM].set(Wp[0, :DIM])
    return _tc_mlp(gu, gi, a, b, b1.reshape(1, -1), W2.T, b2.reshape(1, -1),
                   wpm, Wp[:, DIM:], bp.reshape(1, 1))
